# baseline (speedup 1.0000x reference)
import jax
import jax.numpy as jnp
from jax.experimental import pallas as pl
from jax.experimental.pallas import tpu as pltpu

B, S, N_CLASSES, D = 1024, 50, 1000, 128


BS = 5


def _matmul_body(x_ref, w_ref, o_ref):
    w = w_ref[...].astype(jnp.bfloat16)
    for s in range(BS):
        xs = x_ref[s]
        o_ref[s] = jax.lax.dot_general(
            xs.astype(jnp.bfloat16),
            w,
            (((0,), (0,)), ((), ())),
            preferred_element_type=jnp.float32,
        )


def kernel(x_multi_hot, embedding_weight):
    x_t = jnp.transpose(x_multi_hot, (1, 2, 0))
    out_t = pl.pallas_call(
        _matmul_body,
        grid=(S // BS,),
        in_specs=[
            pl.BlockSpec((BS, N_CLASSES, B), lambda i: (i, 0, 0)),
            pl.BlockSpec((N_CLASSES, D), lambda i: (0, 0)),
        ],
        out_specs=pl.BlockSpec((BS, B, D), lambda i: (i, 0, 0)),
        out_shape=jax.ShapeDtypeStruct((S, B, D), jnp.float32),
        compiler_params=pltpu.CompilerParams(
            dimension_semantics=("parallel",),
        ),
    )(x_t, embedding_weight)
    return jnp.transpose(out_t, (1, 0, 2))


# speedup vs baseline: 1.0150x; 1.0150x over previous
import jax
import jax.numpy as jnp
from jax.experimental import pallas as pl
from jax.experimental.pallas import tpu as pltpu

B, S, N_CLASSES, D = 1024, 50, 1000, 128


BS = 2


def _matmul_body(x_ref, w_ref, o_ref):
    w = w_ref[...].astype(jnp.bfloat16)
    for s in range(BS):
        xs = x_ref[s]
        o_ref[s] = jax.lax.dot_general(
            xs.astype(jnp.bfloat16),
            w,
            (((0,), (0,)), ((), ())),
            preferred_element_type=jnp.float32,
        )


def kernel(x_multi_hot, embedding_weight):
    x_t = jnp.transpose(x_multi_hot, (1, 2, 0))
    out_t = pl.pallas_call(
        _matmul_body,
        grid=(S // BS,),
        in_specs=[
            pl.BlockSpec((BS, N_CLASSES, B), lambda i: (i, 0, 0)),
            pl.BlockSpec((N_CLASSES, D), lambda i: (0, 0)),
        ],
        out_specs=pl.BlockSpec((BS, B, D), lambda i: (i, 0, 0)),
        out_shape=jax.ShapeDtypeStruct((S, B, D), jnp.float32),
        compiler_params=pltpu.CompilerParams(
            dimension_semantics=("parallel",),
        ),
    )(x_t, embedding_weight)
    return jnp.transpose(out_t, (1, 0, 2))


# speedup vs baseline: 1.0184x; 1.0034x over previous
import jax
import jax.numpy as jnp
from jax.experimental import pallas as pl
from jax.experimental.pallas import tpu as pltpu

B, S, N_CLASSES, D = 1024, 50, 1000, 128


BS = 2


HB = B // 2


def _matmul_body(xa_ref, xb_ref, w_ref, o_ref):
    w = w_ref[...].astype(jnp.bfloat16)
    for s in range(BS):
        for h, x_ref in enumerate((xa_ref, xb_ref)):
            xs = x_ref[s]
            o_ref[s, h * HB:(h + 1) * HB, :] = jax.lax.dot_general(
                xs.astype(jnp.bfloat16),
                w,
                (((0,), (0,)), ((), ())),
                preferred_element_type=jnp.float32,
            )


def kernel(x_multi_hot, embedding_weight):
    x_t = jnp.transpose(x_multi_hot, (1, 2, 0))
    out_t = pl.pallas_call(
        _matmul_body,
        grid=(S // BS,),
        in_specs=[
            pl.BlockSpec((BS, N_CLASSES, HB), lambda i: (i, 0, 0)),
            pl.BlockSpec((BS, N_CLASSES, HB), lambda i: (i, 0, 1)),
            pl.BlockSpec((N_CLASSES, D), lambda i: (0, 0)),
        ],
        out_specs=pl.BlockSpec((BS, B, D), lambda i: (i, 0, 0)),
        out_shape=jax.ShapeDtypeStruct((S, B, D), jnp.float32),
        compiler_params=pltpu.CompilerParams(
            dimension_semantics=("parallel",),
        ),
    )(x_t, x_t, embedding_weight)
    return jnp.transpose(out_t, (1, 0, 2))
